# Initial kernel scaffold; baseline (speedup 1.0000x reference)
import jax
import jax.numpy as jnp
from jax.experimental import pallas as pl


def kernel(user_id, movie_id, user_embeddings, movie_embeddings, W0, b0, W1, b1, W2, b2):
    raise NotImplementedError("write your pallas kernel here")


# speedup vs baseline: 2.0705x; 2.0705x over previous
import functools

import jax
import jax.numpy as jnp
from jax import lax
from jax.experimental import pallas as pl
from jax.experimental.pallas import tpu as pltpu
from jax.experimental.pallas import tpu_sc as plsc

VOCAB_ = 10000
EMB_ = 32
BATCH_ = 16384

_NC = 2
_NS = 16
_NW = _NC * _NS
_BPW = BATCH_ // _NW
_CH = 128
_NCHUNK = _BPW // _CH


def _gather_body(uid_ref, mid_ref, utab_ref, mtab_ref, out_u_ref, out_m_ref,
                 uidx, midx, urows, mrows, usem, msem):
    wid = lax.axis_index("s") * _NC + lax.axis_index("c")
    base = wid * _BPW
    row0 = wid * _NCHUNK
    pltpu.sync_copy(uid_ref.at[pl.ds(row0, _NCHUNK)], uidx)
    pltpu.sync_copy(mid_ref.at[pl.ds(row0, _NCHUNK)], midx)
    ucopies = [
        pltpu.async_copy(utab_ref.at[uidx.at[j]],
                         urows.at[pl.ds(j * _CH, _CH)], usem)
        for j in range(_NCHUNK)
    ]
    mcopies = [
        pltpu.async_copy(mtab_ref.at[midx.at[j]],
                         mrows.at[pl.ds(j * _CH, _CH)], msem)
        for j in range(_NCHUNK)
    ]
    for c in ucopies:
        c.wait()
    pltpu.sync_copy(urows, out_u_ref.at[pl.ds(base, _BPW)])
    for c in mcopies:
        c.wait()
    pltpu.sync_copy(mrows, out_m_ref.at[pl.ds(base, _BPW)])


_gather = pl.kernel(
    _gather_body,
    mesh=plsc.VectorSubcoreMesh(core_axis_name="c", subcore_axis_name="s"),
    out_type=[
        jax.ShapeDtypeStruct((BATCH_, EMB_), jnp.float32),
        jax.ShapeDtypeStruct((BATCH_, EMB_), jnp.float32),
    ],
    scratch_types=[
        pltpu.VMEM((_NCHUNK, _CH), jnp.int32),
        pltpu.VMEM((_NCHUNK, _CH), jnp.int32),
        pltpu.VMEM((_BPW, EMB_), jnp.float32),
        pltpu.VMEM((_BPW, EMB_), jnp.float32),
        pltpu.SemaphoreType.DMA,
        pltpu.SemaphoreType.DMA,
    ],
    compiler_params=pltpu.CompilerParams(use_tc_tiling_on_sc=False),
)

_BT = 1024


def _mlp_body(xu_ref, xm_ref, w0u_ref, w0m_ref, b0_ref, w1_ref, b1_ref,
              w2t_ref, b2_ref, out_ref):
    h = jnp.dot(xu_ref[...], w0u_ref[...], preferred_element_type=jnp.float32)
    h = h + jnp.dot(xm_ref[...], w0m_ref[...],
                    preferred_element_type=jnp.float32)
    h = jnp.maximum(h + b0_ref[...], 0.0)
    h = jnp.dot(h, w1_ref[...], preferred_element_type=jnp.float32)
    h = jnp.maximum(h + b1_ref[...], 0.0)
    out_ref[...] = jnp.sum(h * w2t_ref[...], axis=1, keepdims=True) + b2_ref[...]


def _mlp(xu, xm, w0u, w0m, b0, w1, b1, w2t, b2):
    full = lambda r, c: pl.BlockSpec((r, c), lambda i: (0, 0))
    return pl.pallas_call(
        _mlp_body,
        grid=(BATCH_ // _BT,),
        in_specs=[
            pl.BlockSpec((_BT, EMB_), lambda i: (i, 0)),
            pl.BlockSpec((_BT, EMB_), lambda i: (i, 0)),
            full(EMB_, 256),
            full(EMB_, 256),
            full(1, 256),
            full(256, 64),
            full(1, 64),
            full(1, 64),
            full(1, 1),
        ],
        out_specs=pl.BlockSpec((_BT, 1), lambda i: (i, 0)),
        out_shape=jax.ShapeDtypeStruct((BATCH_, 1), jnp.float32),
    )(xu, xm, w0u, w0m, b0, w1, b1, w2t, b2)


def kernel(user_id, movie_id, user_embeddings, movie_embeddings,
           W0, b0, W1, b1, W2, b2):
    uid = user_id.astype(jnp.int32).reshape(_NW * _NCHUNK, _CH)
    mid = movie_id.astype(jnp.int32).reshape(_NW * _NCHUNK, _CH)
    xu, xm = _gather(uid, mid, user_embeddings, movie_embeddings)
    out = _mlp(xu, xm,
               W0[:EMB_], W0[EMB_:], b0.reshape(1, 256),
               W1, b1.reshape(1, 64),
               W2.reshape(1, 64), b2.reshape(1, 1))
    return out.reshape(-1)
